# Initial kernel scaffold; baseline (speedup 1.0000x reference)
import functools

import jax
import jax.numpy as jnp
from jax import lax
from jax.experimental import pallas as pl
from jax.experimental.pallas import tpu as pltpu
from jax.experimental.pallas import tpu_sc as plsc

N = 10000
E = 320000
D = 128
H = 64
O = 2

NC = 2
NS = 16
NW = NC * NS
EPW = E // NW
C = 80
NCHUNK = EPW // C
RPT = N // NS
ZR = 125
CW = 16

RB = 1000
GRID = N // RB

_f32 = jnp.float32


def _proj_body(x_ref, w_ref, b_ref, oa_ref, ob_ref):
    r = jnp.dot(x_ref[...], w_ref[...], preferred_element_type=_f32) + b_ref[...]
    oa_ref[...] = r[:, :H]
    ob_ref[...] = r[:, H:]


def _make_proj(din):
    return pl.pallas_call(
        _proj_body,
        grid=(GRID,),
        in_specs=[
            pl.BlockSpec((RB, din), lambda i: (i, 0)),
            pl.BlockSpec((din, 2 * H), lambda i: (0, 0)),
            pl.BlockSpec((1, 2 * H), lambda i: (0, 0)),
        ],
        out_specs=[pl.BlockSpec((RB, H), lambda i: (i, 0))] * 2,
        out_shape=[jax.ShapeDtypeStruct((N, H), _f32)] * 2,
        name="edgeconv_proj",
    )


def _mid_body(parts_ref, cnts_ref, wb_ref, bb_ref, w2_ref, b2_ref,
              oa_ref, ob_ref):
    s = parts_ref[0] + parts_ref[1]
    c = cnts_ref[0, :, :1] + cnts_ref[1, :, :1]
    inv = 1.0 / jnp.maximum(c, 1.0)
    ind = jnp.minimum(c, 1.0)
    mean = jnp.dot(s * inv, wb_ref[...], preferred_element_type=_f32) + ind * bb_ref[...]
    h = jnp.maximum(mean, 0.0)
    r = jnp.dot(h, w2_ref[...], preferred_element_type=_f32) + b2_ref[...]
    oa_ref[...] = r[:, :H]
    ob_ref[...] = r[:, H:]


_mid = pl.pallas_call(
    _mid_body,
    grid=(GRID,),
    in_specs=[
        pl.BlockSpec((NC, RB, H), lambda i: (0, i, 0)),
        pl.BlockSpec((NC, RB, CW), lambda i: (0, i, 0)),
        pl.BlockSpec((H, H), lambda i: (0, 0)),
        pl.BlockSpec((1, H), lambda i: (0, 0)),
        pl.BlockSpec((H, 2 * H), lambda i: (0, 0)),
        pl.BlockSpec((1, 2 * H), lambda i: (0, 0)),
    ],
    out_specs=[pl.BlockSpec((RB, H), lambda i: (i, 0))] * 2,
    out_shape=[jax.ShapeDtypeStruct((N, H), _f32)] * 2,
    name="edgeconv_mid",
)


def _head_body(parts_ref, cnts_ref, wb_ref, bb_ref, wl_ref, bl_ref, o_ref):
    s = parts_ref[0] + parts_ref[1]
    c = cnts_ref[0, :, :1] + cnts_ref[1, :, :1]
    inv = 1.0 / jnp.maximum(c, 1.0)
    ind = jnp.minimum(c, 1.0)
    mean = jnp.dot(s * inv, wb_ref[...], preferred_element_type=_f32) + ind * bb_ref[...]
    h = jnp.maximum(mean, 0.0)
    o_ref[...] = jnp.dot(h, wl_ref[...], preferred_element_type=_f32) + bl_ref[...]


_head = pl.pallas_call(
    _head_body,
    grid=(GRID,),
    in_specs=[
        pl.BlockSpec((NC, RB, H), lambda i: (0, i, 0)),
        pl.BlockSpec((NC, RB, CW), lambda i: (0, i, 0)),
        pl.BlockSpec((H, H), lambda i: (0, 0)),
        pl.BlockSpec((1, H), lambda i: (0, 0)),
        pl.BlockSpec((H, D), lambda i: (0, 0)),
        pl.BlockSpec((1, D), lambda i: (0, 0)),
    ],
    out_specs=pl.BlockSpec((RB, D), lambda i: (i, 0)),
    out_shape=jax.ShapeDtypeStruct((N, D), _f32),
    name="edgeconv_head",
)


def _edge_body(with_counts, *refs):
    if with_counts:
        (src_h, dst_h, a_h, b_h, parts_h, cnts_h,
         idx_s, idx_d, buf_a, buf_b, zrows, z16, ones_v,
         acc_sh, cnt_sh, sem_a, sem_b) = refs
    else:
        (src_h, dst_h, a_h, b_h, parts_h,
         idx_s, idx_d, buf_a, buf_b, zrows,
         acc_sh, sem_a, sem_b) = refs

    cid = lax.axis_index("c")
    sid = lax.axis_index("s")
    wid = cid * NS + sid

    zv = jnp.zeros((16,), _f32)

    def zrow(r, carry):
        for c4 in range(H // 16):
            zrows[r, pl.ds(c4 * 16, 16)] = zv
        return carry

    lax.fori_loop(0, ZR, zrow, 0)

    for p in range(RPT // ZR):
        pltpu.sync_copy(zrows, acc_sh.at[pl.ds(sid * RPT + p * ZR, ZR)])

    if with_counts:
        def z16row(r, carry):
            z16[r, pl.ds(0, 16)] = zv
            return carry
        lax.fori_loop(0, ZR, z16row, 0)
        for p in range(RPT // ZR):
            pltpu.sync_copy(z16, cnt_sh.at[pl.ds(sid * RPT + p * ZR, ZR)])
        one = jnp.ones((16,), _f32)

        def onerow(r, carry):
            ones_v[r, pl.ds(0, 16)] = one
            return carry
        lax.fori_loop(0, C, onerow, 0)

    plsc.subcore_barrier()

    ebase = wid * EPW

    def chunk(t, carry):
        eb = ebase + t * C
        pltpu.sync_copy(src_h.at[pl.ds(eb, C)], idx_s)
        pltpu.sync_copy(dst_h.at[pl.ds(eb, C)], idx_d)
        cp_a = pltpu.async_copy(a_h.at[idx_d], buf_a, sem_a)
        cp_b = pltpu.async_copy(b_h.at[idx_s], buf_b, sem_b)
        cp_a.wait()
        cp_b.wait()

        def row(r, rcarry):
            for c4 in range(H // 16):
                sl = pl.ds(c4 * 16, 16)
                buf_a[r, sl] = jnp.maximum(buf_a[r, sl] + buf_b[r, sl], 0.0)
            return rcarry

        lax.fori_loop(0, C, row, 0)
        pltpu.sync_copy(buf_a, acc_sh.at[idx_d], add=True)
        if with_counts:
            pltpu.sync_copy(ones_v, cnt_sh.at[idx_d], add=True)
        return carry

    lax.fori_loop(0, NCHUNK, chunk, 0)

    plsc.subcore_barrier()

    pltpu.sync_copy(acc_sh.at[pl.ds(sid * RPT, RPT)],
                    parts_h.at[cid, pl.ds(sid * RPT, RPT)])
    if with_counts:
        pltpu.sync_copy(cnt_sh.at[pl.ds(sid * RPT, RPT)],
                        cnts_h.at[cid, pl.ds(sid * RPT, RPT)])


_sc_mesh = plsc.VectorSubcoreMesh(core_axis_name="c", subcore_axis_name="s")

_edge_pass1 = pl.kernel(
    functools.partial(_edge_body, True),
    out_type=[
        jax.ShapeDtypeStruct((NC, N, H), _f32),
        jax.ShapeDtypeStruct((NC, N, CW), _f32),
    ],
    mesh=_sc_mesh,
    scratch_types=[
        pltpu.VMEM((C,), jnp.int32),
        pltpu.VMEM((C,), jnp.int32),
        pltpu.VMEM((C, H), _f32),
        pltpu.VMEM((C, H), _f32),
        pltpu.VMEM((ZR, H), _f32),
        pltpu.VMEM((ZR, CW), _f32),
        pltpu.VMEM((C, CW), _f32),
        pltpu.VMEM_SHARED((N, H), _f32),
        pltpu.VMEM_SHARED((N, CW), _f32),
        pltpu.SemaphoreType.DMA,
        pltpu.SemaphoreType.DMA,
    ],
    name="edge_pass_l1",
)

_edge_pass2 = pl.kernel(
    functools.partial(_edge_body, False),
    out_type=jax.ShapeDtypeStruct((NC, N, H), _f32),
    mesh=_sc_mesh,
    scratch_types=[
        pltpu.VMEM((C,), jnp.int32),
        pltpu.VMEM((C,), jnp.int32),
        pltpu.VMEM((C, H), _f32),
        pltpu.VMEM((C, H), _f32),
        pltpu.VMEM((ZR, H), _f32),
        pltpu.VMEM_SHARED((N, H), _f32),
        pltpu.SemaphoreType.DMA,
        pltpu.SemaphoreType.DMA,
    ],
    name="edge_pass_l2",
)


def kernel(x, edge_index, W1a, b1a, W1b, b1b, W2a, b2a, W2b, b2b, Wl, bl):
    src = edge_index[0].astype(jnp.int32)
    dst = edge_index[1].astype(jnp.int32)

    w1 = jnp.concatenate([W1a[:D] - W1a[D:], W1a[D:]], axis=1)
    bias1 = jnp.concatenate([b1a, jnp.zeros_like(b1a)])[None]
    w2 = jnp.concatenate([W2a[:H] - W2a[H:], W2a[H:]], axis=1)
    bias2 = jnp.concatenate([b2a, jnp.zeros_like(b2a)])[None]
    wl_pad = jnp.zeros((H, D), _f32).at[:, :O].set(Wl)
    bl_pad = jnp.zeros((1, D), _f32).at[0, :O].set(bl)

    a1, b1v = _make_proj(D)(x, w1, bias1)
    parts1, cnts = _edge_pass1(src, dst, a1, b1v)
    a2, b2v = _mid(parts1, cnts, W1b, b1b[None], w2, bias2)
    parts2 = _edge_pass2(src, dst, a2, b2v)
    out = _head(parts2, cnts, W2b, b2b[None], wl_pad, bl_pad)
    return out[:, :O]


# speedup vs baseline: 7.1886x; 7.1886x over previous
import functools

import jax
import jax.numpy as jnp
from jax import lax
from jax.experimental import pallas as pl
from jax.experimental.pallas import tpu as pltpu
from jax.experimental.pallas import tpu_sc as plsc

N = 10000
E = 320000
D = 128
H = 64
O = 2

NC = 2
NS = 16
NW = NC * NS
EPW = E // NW
C = 80
NCHUNK = EPW // C
RPT = N // NS
RO = 624
ZR = 125
CW = 16

RB = 1000
GRID = N // RB

_f32 = jnp.float32


def _proj_body(x_ref, w_ref, b_ref, oa_ref, ob_ref):
    r = jnp.dot(x_ref[...], w_ref[...], preferred_element_type=_f32) + b_ref[...]
    oa_ref[...] = r[:, :H]
    ob_ref[...] = r[:, H:]


def _make_proj(din):
    return pl.pallas_call(
        _proj_body,
        grid=(GRID,),
        in_specs=[
            pl.BlockSpec((RB, din), lambda i: (i, 0)),
            pl.BlockSpec((din, 2 * H), lambda i: (0, 0)),
            pl.BlockSpec((1, 2 * H), lambda i: (0, 0)),
        ],
        out_specs=[pl.BlockSpec((RB, H), lambda i: (i, 0))] * 2,
        out_shape=[jax.ShapeDtypeStruct((N, H), _f32)] * 2,
        name="edgeconv_proj",
    )


def _mid_body(parts_ref, cnts_ref, wb_ref, bb_ref, w2_ref, b2_ref,
              oa_ref, ob_ref):
    s = parts_ref[0] + parts_ref[1]
    c = cnts_ref[0, :, :1] + cnts_ref[1, :, :1]
    inv = 1.0 / jnp.maximum(c, 1.0)
    ind = jnp.minimum(c, 1.0)
    mean = jnp.dot(s * inv, wb_ref[...], preferred_element_type=_f32) + ind * bb_ref[...]
    h = jnp.maximum(mean, 0.0)
    r = jnp.dot(h, w2_ref[...], preferred_element_type=_f32) + b2_ref[...]
    oa_ref[...] = r[:, :H]
    ob_ref[...] = r[:, H:]


_mid = pl.pallas_call(
    _mid_body,
    grid=(GRID,),
    in_specs=[
        pl.BlockSpec((NC, RB, H), lambda i: (0, i, 0)),
        pl.BlockSpec((NC, RB, CW), lambda i: (0, i, 0)),
        pl.BlockSpec((H, H), lambda i: (0, 0)),
        pl.BlockSpec((1, H), lambda i: (0, 0)),
        pl.BlockSpec((H, 2 * H), lambda i: (0, 0)),
        pl.BlockSpec((1, 2 * H), lambda i: (0, 0)),
    ],
    out_specs=[pl.BlockSpec((RB, H), lambda i: (i, 0))] * 2,
    out_shape=[jax.ShapeDtypeStruct((N, H), _f32)] * 2,
    name="edgeconv_mid",
)


def _head_body(parts_ref, cnts_ref, wb_ref, bb_ref, wl_ref, bl_ref, o_ref):
    s = parts_ref[0] + parts_ref[1]
    c = cnts_ref[0, :, :1] + cnts_ref[1, :, :1]
    inv = 1.0 / jnp.maximum(c, 1.0)
    ind = jnp.minimum(c, 1.0)
    mean = jnp.dot(s * inv, wb_ref[...], preferred_element_type=_f32) + ind * bb_ref[...]
    h = jnp.maximum(mean, 0.0)
    o_ref[...] = jnp.dot(h, wl_ref[...], preferred_element_type=_f32) + bl_ref[...]


_head = pl.pallas_call(
    _head_body,
    grid=(GRID,),
    in_specs=[
        pl.BlockSpec((NC, RB, H), lambda i: (0, i, 0)),
        pl.BlockSpec((NC, RB, CW), lambda i: (0, i, 0)),
        pl.BlockSpec((H, H), lambda i: (0, 0)),
        pl.BlockSpec((1, H), lambda i: (0, 0)),
        pl.BlockSpec((H, D), lambda i: (0, 0)),
        pl.BlockSpec((1, D), lambda i: (0, 0)),
    ],
    out_specs=pl.BlockSpec((RB, D), lambda i: (i, 0)),
    out_shape=jax.ShapeDtypeStruct((N, D), _f32),
    name="edgeconv_head",
)


def _edge_body(with_counts, *refs):
    if with_counts:
        (src_h, dst_h, a_h, b_h, parts_h, cnts_h,
         idx_s, idx_d, buf_a, buf_b, zrows, z16, ones_v,
         acc_sh, cnt_sh, sem_a, sem_b) = refs
    else:
        (src_h, dst_h, a_h, b_h, parts_h,
         idx_s, idx_d, buf_a, buf_b, zrows,
         acc_sh, sem_a, sem_b) = refs

    cid = lax.axis_index("c")
    sid = lax.axis_index("s")
    wid = cid * NS + sid

    zv = jnp.zeros((16,), _f32)

    def zrow(r, carry):
        for c4 in range(H // 16):
            zrows[r, pl.ds(c4 * 16, 16)] = zv
        return carry

    lax.fori_loop(0, ZR, zrow, 0)

    for p in range(RPT // ZR):
        pltpu.sync_copy(zrows, acc_sh.at[pl.ds(sid * RPT + p * ZR, ZR)])

    if with_counts:
        def z16row(r, carry):
            z16[r, pl.ds(0, 16)] = zv
            return carry
        lax.fori_loop(0, ZR, z16row, 0)
        for p in range(RPT // ZR):
            pltpu.sync_copy(z16, cnt_sh.at[pl.ds(sid * RPT + p * ZR, ZR)])
        one = jnp.ones((16,), _f32)

        def onerow(r, carry):
            ones_v[r, pl.ds(0, 16)] = one
            return carry
        lax.fori_loop(0, C, onerow, 0)

    plsc.subcore_barrier()

    ebase = wid * EPW

    def chunk(t, carry):
        eb = ebase + t * C
        pltpu.sync_copy(src_h.at[pl.ds(eb, C)], idx_s)
        pltpu.sync_copy(dst_h.at[pl.ds(eb, C)], idx_d)
        cp_a = pltpu.async_copy(a_h.at[idx_d], buf_a, sem_a)
        cp_b = pltpu.async_copy(b_h.at[idx_s], buf_b, sem_b)
        cp_a.wait()
        cp_b.wait()

        def row(r, rcarry):
            for c4 in range(H // 16):
                sl = pl.ds(c4 * 16, 16)
                buf_a[r, sl] = jnp.maximum(buf_a[r, sl] + buf_b[r, sl], 0.0)
            return rcarry

        lax.fori_loop(0, C, row, 0)
        pltpu.sync_copy(buf_a, acc_sh.at[idx_d], add=True)
        if with_counts:
            pltpu.sync_copy(ones_v, cnt_sh.at[idx_d], add=True)
        return carry

    lax.fori_loop(0, NCHUNK, chunk, 0)

    plsc.subcore_barrier()

    ro = sid * RO
    pltpu.sync_copy(acc_sh.at[pl.ds(ro, RO)], parts_h.at[cid, pl.ds(ro, RO)])
    if with_counts:
        pltpu.sync_copy(cnt_sh.at[pl.ds(ro, RO)], cnts_h.at[cid, pl.ds(ro, RO)])

    @pl.when(sid == NS - 1)
    def _tail():
        tb = NS * RO
        pltpu.sync_copy(acc_sh.at[pl.ds(tb, N - NS * RO)],
                        parts_h.at[cid, pl.ds(tb, N - NS * RO)])
        if with_counts:
            pltpu.sync_copy(cnt_sh.at[pl.ds(tb, N - NS * RO)],
                            cnts_h.at[cid, pl.ds(tb, N - NS * RO)])


_sc_mesh = plsc.VectorSubcoreMesh(core_axis_name="c", subcore_axis_name="s")

_edge_pass1 = pl.kernel(
    functools.partial(_edge_body, True),
    out_type=[
        jax.ShapeDtypeStruct((NC, N, H), _f32),
        jax.ShapeDtypeStruct((NC, N, CW), _f32),
    ],
    mesh=_sc_mesh,
    scratch_types=[
        pltpu.VMEM((C,), jnp.int32),
        pltpu.VMEM((C,), jnp.int32),
        pltpu.VMEM((C, H), _f32),
        pltpu.VMEM((C, H), _f32),
        pltpu.VMEM((ZR, H), _f32),
        pltpu.VMEM((ZR, CW), _f32),
        pltpu.VMEM((C, CW), _f32),
        pltpu.VMEM_SHARED((N, H), _f32),
        pltpu.VMEM_SHARED((N, CW), _f32),
        pltpu.SemaphoreType.DMA,
        pltpu.SemaphoreType.DMA,
    ],
    compiler_params=pltpu.CompilerParams(use_tc_tiling_on_sc=False),
    name="edge_pass_l1",
)

_edge_pass2 = pl.kernel(
    functools.partial(_edge_body, False),
    out_type=jax.ShapeDtypeStruct((NC, N, H), _f32),
    mesh=_sc_mesh,
    scratch_types=[
        pltpu.VMEM((C,), jnp.int32),
        pltpu.VMEM((C,), jnp.int32),
        pltpu.VMEM((C, H), _f32),
        pltpu.VMEM((C, H), _f32),
        pltpu.VMEM((ZR, H), _f32),
        pltpu.VMEM_SHARED((N, H), _f32),
        pltpu.SemaphoreType.DMA,
        pltpu.SemaphoreType.DMA,
    ],
    compiler_params=pltpu.CompilerParams(use_tc_tiling_on_sc=False),
    name="edge_pass_l2",
)


def kernel(x, edge_index, W1a, b1a, W1b, b1b, W2a, b2a, W2b, b2b, Wl, bl):
    src = edge_index[0].astype(jnp.int32)
    dst = edge_index[1].astype(jnp.int32)

    w1 = jnp.concatenate([W1a[:D] - W1a[D:], W1a[D:]], axis=1)
    bias1 = jnp.concatenate([b1a, jnp.zeros_like(b1a)])[None]
    w2 = jnp.concatenate([W2a[:H] - W2a[H:], W2a[H:]], axis=1)
    bias2 = jnp.concatenate([b2a, jnp.zeros_like(b2a)])[None]
    wl_pad = jnp.zeros((H, D), _f32).at[:, :O].set(Wl)
    bl_pad = jnp.zeros((1, D), _f32).at[0, :O].set(bl)

    a1, b1v = _make_proj(D)(x, w1, bias1)
    parts1, cnts = _edge_pass1(src, dst, a1, b1v)
    a2, b2v = _mid(parts1, cnts, W1b, b1b[None], w2, bias2)
    parts2 = _edge_pass2(src, dst, a2, b2v)
    out = _head(parts2, cnts, W2b, b2b[None], wl_pad, bl_pad)
    return out[:, :O]


# speedup vs baseline: 14.2865x; 1.9874x over previous
import functools

import jax
import jax.numpy as jnp
from jax import lax
from jax.experimental import pallas as pl
from jax.experimental.pallas import tpu as pltpu
from jax.experimental.pallas import tpu_sc as plsc

N = 10000
E = 320000
D = 128
H = 64
O = 2

NC = 2
NS = 16
NW = NC * NS
EPW = E // NW
C = 80
NCHUNK = EPW // C
RPT = N // NS
RO = 624
ZR = 125
CW = 16

RB = 1000
GRID = N // RB

_f32 = jnp.float32


def _proj_body(x_ref, w_ref, b_ref, oa_ref, ob_ref):
    r = jnp.dot(x_ref[...], w_ref[...], preferred_element_type=_f32) + b_ref[...]
    oa_ref[...] = r[:, :H]
    ob_ref[...] = r[:, H:]


def _make_proj(din):
    return pl.pallas_call(
        _proj_body,
        grid=(GRID,),
        in_specs=[
            pl.BlockSpec((RB, din), lambda i: (i, 0)),
            pl.BlockSpec((din, 2 * H), lambda i: (0, 0)),
            pl.BlockSpec((1, 2 * H), lambda i: (0, 0)),
        ],
        out_specs=[pl.BlockSpec((RB, H), lambda i: (i, 0))] * 2,
        out_shape=[jax.ShapeDtypeStruct((N, H), _f32)] * 2,
        name="edgeconv_proj",
    )


def _mid_body(parts_ref, cnts_ref, wb_ref, bb_ref, w2_ref, b2_ref,
              oa_ref, ob_ref):
    s = parts_ref[0] + parts_ref[1]
    c = cnts_ref[0, :, :1] + cnts_ref[1, :, :1]
    inv = 1.0 / jnp.maximum(c, 1.0)
    ind = jnp.minimum(c, 1.0)
    mean = jnp.dot(s * inv, wb_ref[...], preferred_element_type=_f32) + ind * bb_ref[...]
    h = jnp.maximum(mean, 0.0)
    r = jnp.dot(h, w2_ref[...], preferred_element_type=_f32) + b2_ref[...]
    oa_ref[...] = r[:, :H]
    ob_ref[...] = r[:, H:]


_mid = pl.pallas_call(
    _mid_body,
    grid=(GRID,),
    in_specs=[
        pl.BlockSpec((NC, RB, H), lambda i: (0, i, 0)),
        pl.BlockSpec((NC, RB, CW), lambda i: (0, i, 0)),
        pl.BlockSpec((H, H), lambda i: (0, 0)),
        pl.BlockSpec((1, H), lambda i: (0, 0)),
        pl.BlockSpec((H, 2 * H), lambda i: (0, 0)),
        pl.BlockSpec((1, 2 * H), lambda i: (0, 0)),
    ],
    out_specs=[pl.BlockSpec((RB, H), lambda i: (i, 0))] * 2,
    out_shape=[jax.ShapeDtypeStruct((N, H), _f32)] * 2,
    name="edgeconv_mid",
)


def _head_body(parts_ref, cnts_ref, wb_ref, bb_ref, wl_ref, bl_ref, o_ref):
    s = parts_ref[0] + parts_ref[1]
    c = cnts_ref[0, :, :1] + cnts_ref[1, :, :1]
    inv = 1.0 / jnp.maximum(c, 1.0)
    ind = jnp.minimum(c, 1.0)
    mean = jnp.dot(s * inv, wb_ref[...], preferred_element_type=_f32) + ind * bb_ref[...]
    h = jnp.maximum(mean, 0.0)
    o_ref[...] = jnp.dot(h, wl_ref[...], preferred_element_type=_f32) + bl_ref[...]


_head = pl.pallas_call(
    _head_body,
    grid=(GRID,),
    in_specs=[
        pl.BlockSpec((NC, RB, H), lambda i: (0, i, 0)),
        pl.BlockSpec((NC, RB, CW), lambda i: (0, i, 0)),
        pl.BlockSpec((H, H), lambda i: (0, 0)),
        pl.BlockSpec((1, H), lambda i: (0, 0)),
        pl.BlockSpec((H, D), lambda i: (0, 0)),
        pl.BlockSpec((1, D), lambda i: (0, 0)),
    ],
    out_specs=pl.BlockSpec((RB, D), lambda i: (i, 0)),
    out_shape=jax.ShapeDtypeStruct((N, D), _f32),
    name="edgeconv_head",
)


def _edge_body(with_counts, *refs):
    if with_counts:
        (src_h, dst_h, a_h, b_h, parts_h, cnts_h,
         idx_s2, idx_d2, ba0, ba1, bb0, bb1, bo0, bo1, zrows, z16, ones_v,
         acc_sh, cnt_sh, sga0, sga1, sgb0, sgb1, ssc0, ssc1, scnt) = refs
    else:
        (src_h, dst_h, a_h, b_h, parts_h,
         idx_s2, idx_d2, ba0, ba1, bb0, bb1, bo0, bo1, zrows,
         acc_sh, sga0, sga1, sgb0, sgb1, ssc0, ssc1) = refs
    ba, bb, bo = (ba0, ba1), (bb0, bb1), (bo0, bo1)
    sga, sgb, ssc = (sga0, sga1), (sgb0, sgb1), (ssc0, ssc1)

    cid = lax.axis_index("c")
    sid = lax.axis_index("s")
    wid = cid * NS + sid

    zv = jnp.zeros((16,), _f32)

    def zrow(r, carry):
        for c4 in range(H // 16):
            zrows[r, pl.ds(c4 * 16, 16)] = zv
        return carry

    lax.fori_loop(0, ZR, zrow, 0)

    for p in range(RPT // ZR):
        pltpu.sync_copy(zrows, acc_sh.at[pl.ds(sid * RPT + p * ZR, ZR)])

    if with_counts:
        def z16row(r, carry):
            z16[r, pl.ds(0, 16)] = zv
            return carry
        lax.fori_loop(0, ZR, z16row, 0)
        for p in range(RPT // ZR):
            pltpu.sync_copy(z16, cnt_sh.at[pl.ds(sid * RPT + p * ZR, ZR)])
        one = jnp.ones((16,), _f32)

        def onerow(r, carry):
            ones_v[r, pl.ds(0, 16)] = one
            return carry
        lax.fori_loop(0, C, onerow, 0)

    pltpu.sync_copy(src_h.at[wid], idx_s2)
    pltpu.sync_copy(dst_h.at[wid], idx_d2)

    plsc.subcore_barrier()

    def issue_g(t, b):
        pltpu.async_copy(a_h.at[idx_d2.at[t]], ba[b], sga[b])
        pltpu.async_copy(b_h.at[idx_s2.at[t]], bb[b], sgb[b])

    def wait_g(b):
        pltpu.make_async_copy(a_h.at[idx_d2.at[0]], ba[b], sga[b]).wait()
        pltpu.make_async_copy(b_h.at[idx_s2.at[0]], bb[b], sgb[b]).wait()

    def wait_sc(b):
        pltpu.make_async_copy(bo[b], acc_sh.at[idx_d2.at[0]], ssc[b]).wait()

    def wait_cnt():
        pltpu.make_async_copy(ones_v, cnt_sh.at[idx_d2.at[0]], scnt).wait()

    def compute(b):
        bab, bbb, bob = ba[b], bb[b], bo[b]

        @plsc.parallel_loop(0, C, unroll=4)
        def _rows(r):
            for c4 in range(H // 16):
                sl = pl.ds(c4 * 16, 16)
                bob[r, sl] = jnp.maximum(bab[r, sl] + bbb[r, sl], 0.0)

    def handle(t, b, do_wait_sc, do_wait_cnt, do_issue_next):
        wait_g(b)
        if do_issue_next:
            issue_g(t + 1, 1 - b)
        if do_wait_sc:
            wait_sc(b)
        compute(b)
        pltpu.async_copy(bo[b], acc_sh.at[idx_d2.at[t]], ssc[b], add=True)
        if with_counts:
            if do_wait_cnt:
                wait_cnt()
            pltpu.async_copy(ones_v, cnt_sh.at[idx_d2.at[t]], scnt, add=True)

    issue_g(0, 0)
    handle(0, 0, False, False, True)
    handle(1, 1, False, True, True)

    def steady(t2, carry):
        t = 2 * t2
        handle(t, 0, True, True, True)
        handle(t + 1, 1, True, True, True)
        return carry

    lax.fori_loop(1, NCHUNK // 2, steady, 0)
    handle(NCHUNK - 1, 0, True, True, False)
    wait_sc(1)
    wait_sc(0)
    if with_counts:
        wait_cnt()

    plsc.subcore_barrier()

    ro = sid * RO
    pltpu.sync_copy(acc_sh.at[pl.ds(ro, RO)], parts_h.at[cid, pl.ds(ro, RO)])
    if with_counts:
        pltpu.sync_copy(cnt_sh.at[pl.ds(ro, RO)], cnts_h.at[cid, pl.ds(ro, RO)])

    @pl.when(sid == NS - 1)
    def _tail():
        tb = NS * RO
        pltpu.sync_copy(acc_sh.at[pl.ds(tb, N - NS * RO)],
                        parts_h.at[cid, pl.ds(tb, N - NS * RO)])
        if with_counts:
            pltpu.sync_copy(cnt_sh.at[pl.ds(tb, N - NS * RO)],
                            cnts_h.at[cid, pl.ds(tb, N - NS * RO)])


_sc_mesh = plsc.VectorSubcoreMesh(core_axis_name="c", subcore_axis_name="s")

_edge_pass1 = pl.kernel(
    functools.partial(_edge_body, True),
    out_type=[
        jax.ShapeDtypeStruct((NC, N, H), _f32),
        jax.ShapeDtypeStruct((NC, N, CW), _f32),
    ],
    mesh=_sc_mesh,
    scratch_types=[
        pltpu.VMEM((NCHUNK, C), jnp.int32),
        pltpu.VMEM((NCHUNK, C), jnp.int32),
        pltpu.VMEM((C, H), _f32),
        pltpu.VMEM((C, H), _f32),
        pltpu.VMEM((C, H), _f32),
        pltpu.VMEM((C, H), _f32),
        pltpu.VMEM((C, H), _f32),
        pltpu.VMEM((C, H), _f32),
        pltpu.VMEM((ZR, H), _f32),
        pltpu.VMEM((ZR, CW), _f32),
        pltpu.VMEM((C, CW), _f32),
        pltpu.VMEM_SHARED((N, H), _f32),
        pltpu.VMEM_SHARED((N, CW), _f32),
        pltpu.SemaphoreType.DMA,
        pltpu.SemaphoreType.DMA,
        pltpu.SemaphoreType.DMA,
        pltpu.SemaphoreType.DMA,
        pltpu.SemaphoreType.DMA,
        pltpu.SemaphoreType.DMA,
        pltpu.SemaphoreType.DMA,
    ],
    compiler_params=pltpu.CompilerParams(use_tc_tiling_on_sc=False),
    name="edge_pass_l1",
)

_edge_pass2 = pl.kernel(
    functools.partial(_edge_body, False),
    out_type=jax.ShapeDtypeStruct((NC, N, H), _f32),
    mesh=_sc_mesh,
    scratch_types=[
        pltpu.VMEM((NCHUNK, C), jnp.int32),
        pltpu.VMEM((NCHUNK, C), jnp.int32),
        pltpu.VMEM((C, H), _f32),
        pltpu.VMEM((C, H), _f32),
        pltpu.VMEM((C, H), _f32),
        pltpu.VMEM((C, H), _f32),
        pltpu.VMEM((C, H), _f32),
        pltpu.VMEM((C, H), _f32),
        pltpu.VMEM((ZR, H), _f32),
        pltpu.VMEM_SHARED((N, H), _f32),
        pltpu.SemaphoreType.DMA,
        pltpu.SemaphoreType.DMA,
        pltpu.SemaphoreType.DMA,
        pltpu.SemaphoreType.DMA,
        pltpu.SemaphoreType.DMA,
        pltpu.SemaphoreType.DMA,
    ],
    compiler_params=pltpu.CompilerParams(use_tc_tiling_on_sc=False),
    name="edge_pass_l2",
)


def kernel(x, edge_index, W1a, b1a, W1b, b1b, W2a, b2a, W2b, b2b, Wl, bl):
    src = edge_index[0].astype(jnp.int32).reshape(NW, NCHUNK, C)
    dst = edge_index[1].astype(jnp.int32).reshape(NW, NCHUNK, C)

    w1 = jnp.concatenate([W1a[:D] - W1a[D:], W1a[D:]], axis=1)
    bias1 = jnp.concatenate([b1a, jnp.zeros_like(b1a)])[None]
    w2 = jnp.concatenate([W2a[:H] - W2a[H:], W2a[H:]], axis=1)
    bias2 = jnp.concatenate([b2a, jnp.zeros_like(b2a)])[None]
    wl_pad = jnp.zeros((H, D), _f32).at[:, :O].set(Wl)
    bl_pad = jnp.zeros((1, D), _f32).at[0, :O].set(bl)

    a1, b1v = _make_proj(D)(x, w1, bias1)
    parts1, cnts = _edge_pass1(src, dst, a1, b1v)
    a2, b2v = _mid(parts1, cnts, W1b, b1b[None], w2, bias2)
    parts2 = _edge_pass2(src, dst, a2, b2v)
    out = _head(parts2, cnts, W2b, b2b[None], wl_pad, bl_pad)
    return out[:, :O]


# speedup vs baseline: 16.3475x; 1.1443x over previous
import functools

import numpy as np

import jax
import jax.numpy as jnp
from jax import lax
from jax.experimental import pallas as pl
from jax.experimental.pallas import tpu as pltpu
from jax.experimental.pallas import tpu_sc as plsc

N = 10000
E = 320000
D = 128
H = 64
O = 2

NC = 2
NS = 16
NW = NC * NS
EPW = E // NW
C = 80
NCHUNK = EPW // C
RPT = N // NS
RO = 624
ZR = 125
CW = 16

RB = 1000
GRID = N // RB

_f32 = jnp.float32
_bf16 = jnp.bfloat16

_PERM = np.concatenate(
    [np.concatenate([g * 32 + 2 * np.arange(16), g * 32 + 2 * np.arange(16) + 1])
     for g in range(H // 32)])


def _proj_body(x_ref, w_ref, b_ref, oa_ref, ob_ref):
    r = jnp.dot(x_ref[...], w_ref[...], preferred_element_type=_f32) + b_ref[...]
    oa_ref[...] = r[:, :H].astype(_bf16)
    ob_ref[...] = r[:, H:].astype(_bf16)


def _make_proj(din):
    return pl.pallas_call(
        _proj_body,
        grid=(GRID,),
        in_specs=[
            pl.BlockSpec((RB, din), lambda i: (i, 0)),
            pl.BlockSpec((din, 2 * H), lambda i: (0, 0)),
            pl.BlockSpec((1, 2 * H), lambda i: (0, 0)),
        ],
        out_specs=[pl.BlockSpec((RB, H), lambda i: (i, 0))] * 2,
        out_shape=[jax.ShapeDtypeStruct((N, H), _bf16)] * 2,
        name="edgeconv_proj",
    )


def _mid_body(parts_ref, cnts_ref, wb_ref, bb_ref, w2_ref, b2_ref,
              oa_ref, ob_ref):
    s = parts_ref[0] + parts_ref[1]
    c = cnts_ref[0, :, :1] + cnts_ref[1, :, :1]
    inv = 1.0 / jnp.maximum(c, 1.0)
    ind = jnp.minimum(c, 1.0)
    mean = jnp.dot(s * inv, wb_ref[...], preferred_element_type=_f32) + ind * bb_ref[...]
    h = jnp.maximum(mean, 0.0)
    r = jnp.dot(h, w2_ref[...], preferred_element_type=_f32) + b2_ref[...]
    oa_ref[...] = r[:, :H].astype(_bf16)
    ob_ref[...] = r[:, H:].astype(_bf16)


_mid = pl.pallas_call(
    _mid_body,
    grid=(GRID,),
    in_specs=[
        pl.BlockSpec((NC, RB, H), lambda i: (0, i, 0)),
        pl.BlockSpec((NC, RB, CW), lambda i: (0, i, 0)),
        pl.BlockSpec((H, H), lambda i: (0, 0)),
        pl.BlockSpec((1, H), lambda i: (0, 0)),
        pl.BlockSpec((H, 2 * H), lambda i: (0, 0)),
        pl.BlockSpec((1, 2 * H), lambda i: (0, 0)),
    ],
    out_specs=[pl.BlockSpec((RB, H), lambda i: (i, 0))] * 2,
    out_shape=[jax.ShapeDtypeStruct((N, H), _bf16)] * 2,
    name="edgeconv_mid",
)


def _head_body(parts_ref, cnts_ref, wb_ref, bb_ref, wl_ref, bl_ref, o_ref):
    s = parts_ref[0] + parts_ref[1]
    c = cnts_ref[0, :, :1] + cnts_ref[1, :, :1]
    inv = 1.0 / jnp.maximum(c, 1.0)
    ind = jnp.minimum(c, 1.0)
    mean = jnp.dot(s * inv, wb_ref[...], preferred_element_type=_f32) + ind * bb_ref[...]
    h = jnp.maximum(mean, 0.0)
    o_ref[...] = jnp.dot(h, wl_ref[...], preferred_element_type=_f32) + bl_ref[...]


_head = pl.pallas_call(
    _head_body,
    grid=(GRID,),
    in_specs=[
        pl.BlockSpec((NC, RB, H), lambda i: (0, i, 0)),
        pl.BlockSpec((NC, RB, CW), lambda i: (0, i, 0)),
        pl.BlockSpec((H, H), lambda i: (0, 0)),
        pl.BlockSpec((1, H), lambda i: (0, 0)),
        pl.BlockSpec((H, D), lambda i: (0, 0)),
        pl.BlockSpec((1, D), lambda i: (0, 0)),
    ],
    out_specs=pl.BlockSpec((RB, D), lambda i: (i, 0)),
    out_shape=jax.ShapeDtypeStruct((N, D), _f32),
    name="edgeconv_head",
)


def _edge_body(with_counts, *refs):
    if with_counts:
        (src_h, dst_h, a_h, b_h, parts_h, cnts_h,
         idx_s2, idx_d2, ba0, ba1, bb0, bb1, bo0, bo1, zrows, z16, ones_v,
         acc_sh, cnt_sh, sga0, sga1, sgb0, sgb1, ssc0, ssc1, scnt) = refs
    else:
        (src_h, dst_h, a_h, b_h, parts_h,
         idx_s2, idx_d2, ba0, ba1, bb0, bb1, bo0, bo1, zrows,
         acc_sh, sga0, sga1, sgb0, sgb1, ssc0, ssc1) = refs
    ba, bb, bo = (ba0, ba1), (bb0, bb1), (bo0, bo1)
    sga, sgb, ssc = (sga0, sga1), (sgb0, sgb1), (ssc0, ssc1)

    cid = lax.axis_index("c")
    sid = lax.axis_index("s")
    wid = cid * NS + sid

    zv = jnp.zeros((16,), _f32)

    def zrow(r, carry):
        for c4 in range(H // 16):
            zrows[r, pl.ds(c4 * 16, 16)] = zv
        return carry

    lax.fori_loop(0, ZR, zrow, 0)

    for p in range(RPT // ZR):
        pltpu.sync_copy(zrows, acc_sh.at[pl.ds(sid * RPT + p * ZR, ZR)])

    if with_counts:
        def z16row(r, carry):
            z16[r, pl.ds(0, 16)] = zv
            return carry
        lax.fori_loop(0, ZR, z16row, 0)
        for p in range(RPT // ZR):
            pltpu.sync_copy(z16, cnt_sh.at[pl.ds(sid * RPT + p * ZR, ZR)])
        one = jnp.ones((16,), _f32)

        def onerow(r, carry):
            ones_v[r, pl.ds(0, 16)] = one
            return carry
        lax.fori_loop(0, C, onerow, 0)

    pltpu.sync_copy(src_h.at[wid], idx_s2)
    pltpu.sync_copy(dst_h.at[wid], idx_d2)

    plsc.subcore_barrier()

    def issue_g(t, b):
        pltpu.async_copy(a_h.at[idx_d2.at[t]], ba[b], sga[b])
        pltpu.async_copy(b_h.at[idx_s2.at[t]], bb[b], sgb[b])

    def wait_g(b):
        pltpu.make_async_copy(a_h.at[idx_d2.at[0]], ba[b], sga[b]).wait()
        pltpu.make_async_copy(b_h.at[idx_s2.at[0]], bb[b], sgb[b]).wait()

    def wait_sc(b):
        pltpu.make_async_copy(bo[b], acc_sh.at[idx_d2.at[0]], ssc[b]).wait()

    def wait_cnt():
        pltpu.make_async_copy(ones_v, cnt_sh.at[idx_d2.at[0]], scnt).wait()

    def compute(b):
        bab, bbb, bob = ba[b], bb[b], bo[b]

        @plsc.parallel_loop(0, C, unroll=4)
        def _rows(r):
            for g in range(H // 32):
                sl32 = pl.ds(g * 32, 32)
                ae, ao = plsc.unpack(bab[r, sl32], format=plsc.PackFormat.INTERLEAVED)
                be, bo_ = plsc.unpack(bbb[r, sl32], format=plsc.PackFormat.INTERLEAVED)
                bob[r, pl.ds(g * 32, 16)] = jnp.maximum(ae + be, 0.0)
                bob[r, pl.ds(g * 32 + 16, 16)] = jnp.maximum(ao + bo_, 0.0)

    def handle(t, b, do_wait_sc, do_wait_cnt, do_issue_next):
        wait_g(b)
        if do_issue_next:
            issue_g(t + 1, 1 - b)
        if do_wait_sc:
            wait_sc(b)
        compute(b)
        pltpu.async_copy(bo[b], acc_sh.at[idx_d2.at[t]], ssc[b], add=True)
        if with_counts:
            if do_wait_cnt:
                wait_cnt()
            pltpu.async_copy(ones_v, cnt_sh.at[idx_d2.at[t]], scnt, add=True)

    issue_g(0, 0)
    handle(0, 0, False, False, True)
    handle(1, 1, False, True, True)

    def steady(t2, carry):
        t = 2 * t2
        handle(t, 0, True, True, True)
        handle(t + 1, 1, True, True, True)
        return carry

    lax.fori_loop(1, NCHUNK // 2, steady, 0)
    handle(NCHUNK - 1, 0, True, True, False)
    wait_sc(1)
    wait_sc(0)
    if with_counts:
        wait_cnt()

    plsc.subcore_barrier()

    ro = sid * RO
    pltpu.sync_copy(acc_sh.at[pl.ds(ro, RO)], parts_h.at[cid, pl.ds(ro, RO)])
    if with_counts:
        pltpu.sync_copy(cnt_sh.at[pl.ds(ro, RO)], cnts_h.at[cid, pl.ds(ro, RO)])

    @pl.when(sid == NS - 1)
    def _tail():
        tb = NS * RO
        pltpu.sync_copy(acc_sh.at[pl.ds(tb, N - NS * RO)],
                        parts_h.at[cid, pl.ds(tb, N - NS * RO)])
        if with_counts:
            pltpu.sync_copy(cnt_sh.at[pl.ds(tb, N - NS * RO)],
                            cnts_h.at[cid, pl.ds(tb, N - NS * RO)])


_sc_mesh = plsc.VectorSubcoreMesh(core_axis_name="c", subcore_axis_name="s")

_edge_pass1 = pl.kernel(
    functools.partial(_edge_body, True),
    out_type=[
        jax.ShapeDtypeStruct((NC, N, H), _f32),
        jax.ShapeDtypeStruct((NC, N, CW), _f32),
    ],
    mesh=_sc_mesh,
    scratch_types=[
        pltpu.VMEM((NCHUNK, C), jnp.int32),
        pltpu.VMEM((NCHUNK, C), jnp.int32),
        pltpu.VMEM((C, H), _bf16),
        pltpu.VMEM((C, H), _bf16),
        pltpu.VMEM((C, H), _bf16),
        pltpu.VMEM((C, H), _bf16),
        pltpu.VMEM((C, H), _f32),
        pltpu.VMEM((C, H), _f32),
        pltpu.VMEM((ZR, H), _f32),
        pltpu.VMEM((ZR, CW), _f32),
        pltpu.VMEM((C, CW), _f32),
        pltpu.VMEM_SHARED((N, H), _f32),
        pltpu.VMEM_SHARED((N, CW), _f32),
        pltpu.SemaphoreType.DMA,
        pltpu.SemaphoreType.DMA,
        pltpu.SemaphoreType.DMA,
        pltpu.SemaphoreType.DMA,
        pltpu.SemaphoreType.DMA,
        pltpu.SemaphoreType.DMA,
        pltpu.SemaphoreType.DMA,
    ],
    compiler_params=pltpu.CompilerParams(use_tc_tiling_on_sc=False, needs_layout_passes=False),
    name="edge_pass_l1",
)

_edge_pass2 = pl.kernel(
    functools.partial(_edge_body, False),
    out_type=jax.ShapeDtypeStruct((NC, N, H), _f32),
    mesh=_sc_mesh,
    scratch_types=[
        pltpu.VMEM((NCHUNK, C), jnp.int32),
        pltpu.VMEM((NCHUNK, C), jnp.int32),
        pltpu.VMEM((C, H), _bf16),
        pltpu.VMEM((C, H), _bf16),
        pltpu.VMEM((C, H), _bf16),
        pltpu.VMEM((C, H), _bf16),
        pltpu.VMEM((C, H), _f32),
        pltpu.VMEM((C, H), _f32),
        pltpu.VMEM((ZR, H), _f32),
        pltpu.VMEM_SHARED((N, H), _f32),
        pltpu.SemaphoreType.DMA,
        pltpu.SemaphoreType.DMA,
        pltpu.SemaphoreType.DMA,
        pltpu.SemaphoreType.DMA,
        pltpu.SemaphoreType.DMA,
        pltpu.SemaphoreType.DMA,
    ],
    compiler_params=pltpu.CompilerParams(use_tc_tiling_on_sc=False, needs_layout_passes=False),
    name="edge_pass_l2",
)


def kernel(x, edge_index, W1a, b1a, W1b, b1b, W2a, b2a, W2b, b2b, Wl, bl):
    src = edge_index[0].astype(jnp.int32).reshape(NW, NCHUNK, C)
    dst = edge_index[1].astype(jnp.int32).reshape(NW, NCHUNK, C)

    w1 = jnp.concatenate([W1a[:D] - W1a[D:], W1a[D:]], axis=1)
    bias1 = jnp.concatenate([b1a, jnp.zeros_like(b1a)])[None]
    w2 = jnp.concatenate([W2a[:H] - W2a[H:], W2a[H:]], axis=1)
    bias2 = jnp.concatenate([b2a, jnp.zeros_like(b2a)])[None]
    wl_pad = jnp.zeros((H, D), _f32).at[:, :O].set(Wl)
    bl_pad = jnp.zeros((1, D), _f32).at[0, :O].set(bl)

    w1b_p = W1b[_PERM, :]
    w2b_p = W2b[_PERM, :]

    a1, b1v = _make_proj(D)(x, w1, bias1)
    parts1, cnts = _edge_pass1(src, dst, a1, b1v)
    a2, b2v = _mid(parts1, cnts, w1b_p, b1b[None], w2, bias2)
    parts2 = _edge_pass2(src, dst, a2, b2v)
    out = _head(parts2, cnts, w2b_p, b2b[None], wl_pad, bl_pad)
    return out[:, :O]


# speedup vs baseline: 16.4684x; 1.0074x over previous
import functools

import jax
import jax.numpy as jnp
from jax import lax
from jax.experimental import pallas as pl
from jax.experimental.pallas import tpu as pltpu
from jax.experimental.pallas import tpu_sc as plsc

N = 10000
E = 320000
D = 128
H = 64
O = 2

NC = 2
NS = 16
NW = NC * NS
EPW = E // NW
C = 80
NCHUNK = EPW // C
RPT = N // NS
RO = 624
ZR = 125
CW = 16

RB = 1000
GRID = N // RB

_f32 = jnp.float32
_bf16 = jnp.bfloat16


def _proj_body(x_ref, w_ref, b_ref, oa_ref, ob_ref):
    r = jnp.dot(x_ref[...], w_ref[...], preferred_element_type=_f32) + b_ref[...]
    oa_ref[...] = r[:, :H].astype(_bf16)
    ob_ref[...] = r[:, H:].astype(_bf16)


def _make_proj(din):
    return pl.pallas_call(
        _proj_body,
        grid=(GRID,),
        in_specs=[
            pl.BlockSpec((RB, din), lambda i: (i, 0)),
            pl.BlockSpec((din, 2 * H), lambda i: (0, 0)),
            pl.BlockSpec((1, 2 * H), lambda i: (0, 0)),
        ],
        out_specs=[pl.BlockSpec((RB, H), lambda i: (i, 0))] * 2,
        out_shape=[jax.ShapeDtypeStruct((N, H), _bf16)] * 2,
        name="edgeconv_proj",
    )


def _mid_body(parts_ref, cnts_ref, wb_ref, bb_ref, w2_ref, b2_ref,
              oa_ref, ob_ref):
    s = parts_ref[0].astype(_f32) + parts_ref[1].astype(_f32)
    c = cnts_ref[0, :, :1] + cnts_ref[1, :, :1]
    inv = 1.0 / jnp.maximum(c, 1.0)
    ind = jnp.minimum(c, 1.0)
    mean = jnp.dot(s * inv, wb_ref[...], preferred_element_type=_f32) + ind * bb_ref[...]
    h = jnp.maximum(mean, 0.0)
    r = jnp.dot(h, w2_ref[...], preferred_element_type=_f32) + b2_ref[...]
    oa_ref[...] = r[:, :H].astype(_bf16)
    ob_ref[...] = r[:, H:].astype(_bf16)


_mid = pl.pallas_call(
    _mid_body,
    grid=(GRID,),
    in_specs=[
        pl.BlockSpec((NC, RB, H), lambda i: (0, i, 0)),
        pl.BlockSpec((NC, RB, CW), lambda i: (0, i, 0)),
        pl.BlockSpec((H, H), lambda i: (0, 0)),
        pl.BlockSpec((1, H), lambda i: (0, 0)),
        pl.BlockSpec((H, 2 * H), lambda i: (0, 0)),
        pl.BlockSpec((1, 2 * H), lambda i: (0, 0)),
    ],
    out_specs=[pl.BlockSpec((RB, H), lambda i: (i, 0))] * 2,
    out_shape=[jax.ShapeDtypeStruct((N, H), _bf16)] * 2,
    name="edgeconv_mid",
)


def _head_body(parts_ref, cnts_ref, wb_ref, bb_ref, wl_ref, bl_ref, o_ref):
    s = parts_ref[0].astype(_f32) + parts_ref[1].astype(_f32)
    c = cnts_ref[0, :, :1] + cnts_ref[1, :, :1]
    inv = 1.0 / jnp.maximum(c, 1.0)
    ind = jnp.minimum(c, 1.0)
    mean = jnp.dot(s * inv, wb_ref[...], preferred_element_type=_f32) + ind * bb_ref[...]
    h = jnp.maximum(mean, 0.0)
    o_ref[...] = jnp.dot(h, wl_ref[...], preferred_element_type=_f32) + bl_ref[...]


_head = pl.pallas_call(
    _head_body,
    grid=(GRID,),
    in_specs=[
        pl.BlockSpec((NC, RB, H), lambda i: (0, i, 0)),
        pl.BlockSpec((NC, RB, CW), lambda i: (0, i, 0)),
        pl.BlockSpec((H, H), lambda i: (0, 0)),
        pl.BlockSpec((1, H), lambda i: (0, 0)),
        pl.BlockSpec((H, D), lambda i: (0, 0)),
        pl.BlockSpec((1, D), lambda i: (0, 0)),
    ],
    out_specs=pl.BlockSpec((RB, D), lambda i: (i, 0)),
    out_shape=jax.ShapeDtypeStruct((N, D), _f32),
    name="edgeconv_head",
)


def _edge_body(with_counts, *refs):
    if with_counts:
        (src_h, dst_h, a_h, b_h, parts_h, cnts_h,
         idx_s2, idx_d2, ba0, ba1, bb0, bb1, bo0, bo1, zrows, z16, ones_v,
         acc_sh, cnt_sh, sga0, sga1, sgb0, sgb1, ssc0, ssc1, scnt) = refs
    else:
        (src_h, dst_h, a_h, b_h, parts_h,
         idx_s2, idx_d2, ba0, ba1, bb0, bb1, bo0, bo1, zrows,
         acc_sh, sga0, sga1, sgb0, sgb1, ssc0, ssc1) = refs
    ba, bb, bo = (ba0, ba1), (bb0, bb1), (bo0, bo1)
    sga, sgb, ssc = (sga0, sga1), (sgb0, sgb1), (ssc0, ssc1)

    cid = lax.axis_index("c")
    sid = lax.axis_index("s")
    wid = cid * NS + sid

    zv = jnp.zeros((16,), _f32)
    zvh = jnp.zeros((32,), _bf16)

    def zrow(r, carry):
        for c2 in range(H // 32):
            zrows[r, pl.ds(c2 * 32, 32)] = zvh
        return carry

    lax.fori_loop(0, ZR, zrow, 0)

    for p in range(RPT // ZR):
        pltpu.sync_copy(zrows, acc_sh.at[pl.ds(sid * RPT + p * ZR, ZR)])

    if with_counts:
        def z16row(r, carry):
            z16[r, pl.ds(0, 16)] = zv
            return carry
        lax.fori_loop(0, ZR, z16row, 0)
        for p in range(RPT // ZR):
            pltpu.sync_copy(z16, cnt_sh.at[pl.ds(sid * RPT + p * ZR, ZR)])
        one = jnp.ones((16,), _f32)

        def onerow(r, carry):
            ones_v[r, pl.ds(0, 16)] = one
            return carry
        lax.fori_loop(0, C, onerow, 0)

    pltpu.sync_copy(src_h.at[wid], idx_s2)
    pltpu.sync_copy(dst_h.at[wid], idx_d2)

    plsc.subcore_barrier()

    def issue_g(t, b):
        pltpu.async_copy(a_h.at[idx_d2.at[t]], ba[b], sga[b])
        pltpu.async_copy(b_h.at[idx_s2.at[t]], bb[b], sgb[b])

    def wait_g(b):
        pltpu.make_async_copy(a_h.at[idx_d2.at[0]], ba[b], sga[b]).wait()
        pltpu.make_async_copy(b_h.at[idx_s2.at[0]], bb[b], sgb[b]).wait()

    def wait_sc(b):
        pltpu.make_async_copy(bo[b], acc_sh.at[idx_d2.at[0]], ssc[b]).wait()

    def wait_cnt():
        pltpu.make_async_copy(ones_v, cnt_sh.at[idx_d2.at[0]], scnt).wait()

    def compute(b):
        bab, bbb, bob = ba[b], bb[b], bo[b]
        zero = jnp.zeros((32,), _bf16)

        @plsc.parallel_loop(0, C, unroll=4)
        def _rows(r):
            for g in range(H // 32):
                sl32 = pl.ds(g * 32, 32)
                bob[r, sl32] = jnp.maximum(bab[r, sl32] + bbb[r, sl32], zero)

    def handle(t, b, do_wait_sc, do_wait_cnt, do_issue_next):
        wait_g(b)
        if do_issue_next:
            issue_g(t + 1, 1 - b)
        if do_wait_sc:
            wait_sc(b)
        compute(b)
        pltpu.async_copy(bo[b], acc_sh.at[idx_d2.at[t]], ssc[b], add=True)
        if with_counts:
            if do_wait_cnt:
                wait_cnt()
            pltpu.async_copy(ones_v, cnt_sh.at[idx_d2.at[t]], scnt, add=True)

    issue_g(0, 0)
    handle(0, 0, False, False, True)
    handle(1, 1, False, True, True)

    def steady(t2, carry):
        t = 2 * t2
        handle(t, 0, True, True, True)
        handle(t + 1, 1, True, True, True)
        return carry

    lax.fori_loop(1, NCHUNK // 2, steady, 0)
    handle(NCHUNK - 1, 0, True, True, False)
    wait_sc(1)
    wait_sc(0)
    if with_counts:
        wait_cnt()

    plsc.subcore_barrier()

    ro = sid * RO
    pltpu.sync_copy(acc_sh.at[pl.ds(ro, RO)], parts_h.at[cid, pl.ds(ro, RO)])
    if with_counts:
        pltpu.sync_copy(cnt_sh.at[pl.ds(ro, RO)], cnts_h.at[cid, pl.ds(ro, RO)])

    @pl.when(sid == NS - 1)
    def _tail():
        tb = NS * RO
        pltpu.sync_copy(acc_sh.at[pl.ds(tb, N - NS * RO)],
                        parts_h.at[cid, pl.ds(tb, N - NS * RO)])
        if with_counts:
            pltpu.sync_copy(cnt_sh.at[pl.ds(tb, N - NS * RO)],
                            cnts_h.at[cid, pl.ds(tb, N - NS * RO)])


_sc_mesh = plsc.VectorSubcoreMesh(core_axis_name="c", subcore_axis_name="s")

_edge_pass1 = pl.kernel(
    functools.partial(_edge_body, True),
    out_type=[
        jax.ShapeDtypeStruct((NC, N, H), _bf16),
        jax.ShapeDtypeStruct((NC, N, CW), _f32),
    ],
    mesh=_sc_mesh,
    scratch_types=[
        pltpu.VMEM((NCHUNK, C), jnp.int32),
        pltpu.VMEM((NCHUNK, C), jnp.int32),
        pltpu.VMEM((C, H), _bf16),
        pltpu.VMEM((C, H), _bf16),
        pltpu.VMEM((C, H), _bf16),
        pltpu.VMEM((C, H), _bf16),
        pltpu.VMEM((C, H), _bf16),
        pltpu.VMEM((C, H), _bf16),
        pltpu.VMEM((ZR, H), _bf16),
        pltpu.VMEM((ZR, CW), _f32),
        pltpu.VMEM((C, CW), _f32),
        pltpu.VMEM_SHARED((N, H), _bf16),
        pltpu.VMEM_SHARED((N, CW), _f32),
        pltpu.SemaphoreType.DMA,
        pltpu.SemaphoreType.DMA,
        pltpu.SemaphoreType.DMA,
        pltpu.SemaphoreType.DMA,
        pltpu.SemaphoreType.DMA,
        pltpu.SemaphoreType.DMA,
        pltpu.SemaphoreType.DMA,
    ],
    compiler_params=pltpu.CompilerParams(use_tc_tiling_on_sc=False, needs_layout_passes=False),
    name="edge_pass_l1",
)

_edge_pass2 = pl.kernel(
    functools.partial(_edge_body, False),
    out_type=jax.ShapeDtypeStruct((NC, N, H), _bf16),
    mesh=_sc_mesh,
    scratch_types=[
        pltpu.VMEM((NCHUNK, C), jnp.int32),
        pltpu.VMEM((NCHUNK, C), jnp.int32),
        pltpu.VMEM((C, H), _bf16),
        pltpu.VMEM((C, H), _bf16),
        pltpu.VMEM((C, H), _bf16),
        pltpu.VMEM((C, H), _bf16),
        pltpu.VMEM((C, H), _bf16),
        pltpu.VMEM((C, H), _bf16),
        pltpu.VMEM((ZR, H), _bf16),
        pltpu.VMEM_SHARED((N, H), _bf16),
        pltpu.SemaphoreType.DMA,
        pltpu.SemaphoreType.DMA,
        pltpu.SemaphoreType.DMA,
        pltpu.SemaphoreType.DMA,
        pltpu.SemaphoreType.DMA,
        pltpu.SemaphoreType.DMA,
    ],
    compiler_params=pltpu.CompilerParams(use_tc_tiling_on_sc=False, needs_layout_passes=False),
    name="edge_pass_l2",
)


def kernel(x, edge_index, W1a, b1a, W1b, b1b, W2a, b2a, W2b, b2b, Wl, bl):
    src = edge_index[0].astype(jnp.int32).reshape(NW, NCHUNK, C)
    dst = edge_index[1].astype(jnp.int32).reshape(NW, NCHUNK, C)

    w1 = jnp.concatenate([W1a[:D] - W1a[D:], W1a[D:]], axis=1)
    bias1 = jnp.concatenate([b1a, jnp.zeros_like(b1a)])[None]
    w2 = jnp.concatenate([W2a[:H] - W2a[H:], W2a[H:]], axis=1)
    bias2 = jnp.concatenate([b2a, jnp.zeros_like(b2a)])[None]
    wl_pad = jnp.zeros((H, D), _f32).at[:, :O].set(Wl)
    bl_pad = jnp.zeros((1, D), _f32).at[0, :O].set(bl)

    a1, b1v = _make_proj(D)(x, w1, bias1)
    parts1, cnts = _edge_pass1(src, dst, a1, b1v)
    a2, b2v = _mid(parts1, cnts, W1b, b1b[None], w2, bias2)
    parts2 = _edge_pass2(src, dst, a2, b2v)
    out = _head(parts2, cnts, W2b, b2b[None], wl_pad, bl_pad)
    return out[:, :O]


# speedup vs baseline: 19.6164x; 1.1912x over previous
import functools

import numpy as np

import jax
import jax.numpy as jnp
from jax import lax
from jax.experimental import pallas as pl
from jax.experimental.pallas import tpu as pltpu
from jax.experimental.pallas import tpu_sc as plsc

N = 10000
E = 320000
D = 128
H = 64
O = 2

NC = 2
NS = 16
NW = NC * NS
EPW = E // NW
C = 80
NCHUNK = EPW // C
RPT = N // NS
RO = 624
ZR = 125
CW = 16

RB = 1000
GRID = N // RB

_f32 = jnp.float32
_bf16 = jnp.bfloat16

_PERM = np.concatenate(
    [np.concatenate([g * 32 + 2 * np.arange(16), g * 32 + 2 * np.arange(16) + 1])
     for g in range(H // 32)])


def _proj_body(x_ref, w_ref, b_ref, oa_ref, ob_ref):
    r = jnp.dot(x_ref[...], w_ref[...], preferred_element_type=_f32) + b_ref[...]
    oa_ref[...] = r[:, :H].astype(_bf16)
    ob_ref[...] = r[:, H:].astype(_bf16)


def _make_proj(din):
    return pl.pallas_call(
        _proj_body,
        grid=(GRID,),
        in_specs=[
            pl.BlockSpec((RB, din), lambda i: (i, 0)),
            pl.BlockSpec((din, 2 * H), lambda i: (0, 0)),
            pl.BlockSpec((1, 2 * H), lambda i: (0, 0)),
        ],
        out_specs=[pl.BlockSpec((RB, H), lambda i: (i, 0))] * 2,
        out_shape=[jax.ShapeDtypeStruct((N, H), _bf16)] * 2,
        name="edgeconv_proj",
    )


def _mid_body(parts_ref, cnts_ref, wb_ref, bb_ref, w2_ref, b2_ref,
              oa_ref, ob_ref):
    s = parts_ref[0] + parts_ref[1]
    c = cnts_ref[0, :, :1] + cnts_ref[1, :, :1]
    inv = 1.0 / jnp.maximum(c, 1.0)
    ind = jnp.minimum(c, 1.0)
    mean = jnp.dot(s * inv, wb_ref[...], preferred_element_type=_f32) + ind * bb_ref[...]
    h = jnp.maximum(mean, 0.0)
    r = jnp.dot(h, w2_ref[...], preferred_element_type=_f32) + b2_ref[...]
    oa_ref[...] = r[:, :H].astype(_bf16)
    ob_ref[...] = r[:, H:].astype(_bf16)


_mid = pl.pallas_call(
    _mid_body,
    grid=(GRID,),
    in_specs=[
        pl.BlockSpec((NC, RB, H), lambda i: (0, i, 0)),
        pl.BlockSpec((NC, RB, CW), lambda i: (0, i, 0)),
        pl.BlockSpec((H, H), lambda i: (0, 0)),
        pl.BlockSpec((1, H), lambda i: (0, 0)),
        pl.BlockSpec((H, 2 * H), lambda i: (0, 0)),
        pl.BlockSpec((1, 2 * H), lambda i: (0, 0)),
    ],
    out_specs=[pl.BlockSpec((RB, H), lambda i: (i, 0))] * 2,
    out_shape=[jax.ShapeDtypeStruct((N, H), _bf16)] * 2,
    name="edgeconv_mid",
)


def _head_body(parts_ref, cnts_ref, wb_ref, bb_ref, wl_ref, bl_ref, o_ref):
    s = parts_ref[0] + parts_ref[1]
    c = cnts_ref[0, :, :1] + cnts_ref[1, :, :1]
    inv = 1.0 / jnp.maximum(c, 1.0)
    ind = jnp.minimum(c, 1.0)
    mean = jnp.dot(s * inv, wb_ref[...], preferred_element_type=_f32) + ind * bb_ref[...]
    h = jnp.maximum(mean, 0.0)
    o_ref[...] = jnp.dot(h, wl_ref[...], preferred_element_type=_f32) + bl_ref[...]


_head = pl.pallas_call(
    _head_body,
    grid=(GRID,),
    in_specs=[
        pl.BlockSpec((NC, RB, H), lambda i: (0, i, 0)),
        pl.BlockSpec((NC, RB, CW), lambda i: (0, i, 0)),
        pl.BlockSpec((H, H), lambda i: (0, 0)),
        pl.BlockSpec((1, H), lambda i: (0, 0)),
        pl.BlockSpec((H, D), lambda i: (0, 0)),
        pl.BlockSpec((1, D), lambda i: (0, 0)),
    ],
    out_specs=pl.BlockSpec((RB, D), lambda i: (i, 0)),
    out_shape=jax.ShapeDtypeStruct((N, D), _f32),
    name="edgeconv_head",
)


def _edge_body(with_counts, *refs):
    if with_counts:
        (src_h, dst_h, a_h, b_h, parts_h, cnts_h,
         idx_s2, idx_d2, ba0, ba1, bb0, bb1, bo0, bo1, zrows, z16, ones_v,
         acc_sh, cnt_sh, sga0, sga1, sgb0, sgb1, ssc0, ssc1, scnt) = refs
    else:
        (src_h, dst_h, a_h, b_h, parts_h,
         idx_s2, idx_d2, ba0, ba1, bb0, bb1, bo0, bo1, zrows,
         acc_sh, sga0, sga1, sgb0, sgb1, ssc0, ssc1) = refs
    ba, bb, bo = (ba0, ba1), (bb0, bb1), (bo0, bo1)
    sga, sgb, ssc = (sga0, sga1), (sgb0, sgb1), (ssc0, ssc1)

    cid = lax.axis_index("c")
    sid = lax.axis_index("s")
    wid = cid * NS + sid

    zv = jnp.zeros((16,), _f32)

    def zrow(r, carry):
        for c4 in range(H // 16):
            zrows[r, pl.ds(c4 * 16, 16)] = zv
        return carry

    lax.fori_loop(0, ZR, zrow, 0)

    for p in range(RPT // ZR):
        pltpu.sync_copy(zrows, acc_sh.at[pl.ds(sid * RPT + p * ZR, ZR)])

    if with_counts:
        def z16row(r, carry):
            z16[r, pl.ds(0, 16)] = zv
            return carry
        lax.fori_loop(0, ZR, z16row, 0)
        for p in range(RPT // ZR):
            pltpu.sync_copy(z16, cnt_sh.at[pl.ds(sid * RPT + p * ZR, ZR)])
        one = jnp.ones((16,), _f32)

        def onerow(r, carry):
            ones_v[r, pl.ds(0, 16)] = one
            return carry
        lax.fori_loop(0, C, onerow, 0)

    pltpu.sync_copy(src_h.at[wid], idx_s2)
    pltpu.sync_copy(dst_h.at[wid], idx_d2)

    plsc.subcore_barrier()

    def issue_g(t, b):
        pltpu.async_copy(a_h.at[idx_d2.at[t]], ba[b], sga[b])
        pltpu.async_copy(b_h.at[idx_s2.at[t]], bb[b], sgb[b])

    def wait_g(b):
        pltpu.make_async_copy(a_h.at[idx_d2.at[0]], ba[b], sga[b]).wait()
        pltpu.make_async_copy(b_h.at[idx_s2.at[0]], bb[b], sgb[b]).wait()

    def wait_sc(b):
        pltpu.make_async_copy(bo[b], acc_sh.at[idx_d2.at[0]], ssc[b]).wait()

    def wait_cnt():
        pltpu.make_async_copy(ones_v, cnt_sh.at[idx_d2.at[0]], scnt).wait()

    def compute(b):
        bab, bbb, bob = ba[b], bb[b], bo[b]

        @plsc.parallel_loop(0, C, unroll=8)
        def _rows(r):
            for g in range(H // 32):
                sl32 = pl.ds(g * 32, 32)
                ae, ao = plsc.unpack(bab[r, sl32], format=plsc.PackFormat.INTERLEAVED)
                be, bo_ = plsc.unpack(bbb[r, sl32], format=plsc.PackFormat.INTERLEAVED)
                bob[r, pl.ds(g * 32, 16)] = jnp.maximum(ae + be, 0.0)
                bob[r, pl.ds(g * 32 + 16, 16)] = jnp.maximum(ao + bo_, 0.0)

    def handle(t, b, do_wait_sc, do_wait_cnt, do_issue_next):
        if do_issue_next:
            issue_g(t + 1, 1 - b)
        wait_g(b)
        if do_wait_sc:
            wait_sc(b)
        compute(b)
        pltpu.async_copy(bo[b], acc_sh.at[idx_d2.at[t]], ssc[b], add=True)
        if with_counts:
            if do_wait_cnt:
                wait_cnt()
            pltpu.async_copy(ones_v, cnt_sh.at[idx_d2.at[t]], scnt, add=True)

    issue_g(0, 0)
    handle(0, 0, False, False, True)
    handle(1, 1, False, True, True)

    def steady(t2, carry):
        t = 2 * t2
        handle(t, 0, True, True, True)
        handle(t + 1, 1, True, True, True)
        return carry

    lax.fori_loop(1, NCHUNK // 2, steady, 0)
    handle(NCHUNK - 1, 0, True, True, False)
    wait_sc(1)
    wait_sc(0)
    if with_counts:
        wait_cnt()

    plsc.subcore_barrier()

    ro = sid * RO
    pltpu.sync_copy(acc_sh.at[pl.ds(ro, RO)], parts_h.at[cid, pl.ds(ro, RO)])
    if with_counts:
        pltpu.sync_copy(cnt_sh.at[pl.ds(ro, RO)], cnts_h.at[cid, pl.ds(ro, RO)])

    @pl.when(sid == NS - 1)
    def _tail():
        tb = NS * RO
        pltpu.sync_copy(acc_sh.at[pl.ds(tb, N - NS * RO)],
                        parts_h.at[cid, pl.ds(tb, N - NS * RO)])
        if with_counts:
            pltpu.sync_copy(cnt_sh.at[pl.ds(tb, N - NS * RO)],
                            cnts_h.at[cid, pl.ds(tb, N - NS * RO)])


_sc_mesh = plsc.VectorSubcoreMesh(core_axis_name="c", subcore_axis_name="s")

_edge_pass1 = pl.kernel(
    functools.partial(_edge_body, True),
    out_type=[
        jax.ShapeDtypeStruct((NC, N, H), _f32),
        jax.ShapeDtypeStruct((NC, N, CW), _f32),
    ],
    mesh=_sc_mesh,
    scratch_types=[
        pltpu.VMEM((NCHUNK, C), jnp.int32),
        pltpu.VMEM((NCHUNK, C), jnp.int32),
        pltpu.VMEM((C, H), _bf16),
        pltpu.VMEM((C, H), _bf16),
        pltpu.VMEM((C, H), _bf16),
        pltpu.VMEM((C, H), _bf16),
        pltpu.VMEM((C, H), _f32),
        pltpu.VMEM((C, H), _f32),
        pltpu.VMEM((ZR, H), _f32),
        pltpu.VMEM((ZR, CW), _f32),
        pltpu.VMEM((C, CW), _f32),
        pltpu.VMEM_SHARED((N, H), _f32),
        pltpu.VMEM_SHARED((N, CW), _f32),
        pltpu.SemaphoreType.DMA,
        pltpu.SemaphoreType.DMA,
        pltpu.SemaphoreType.DMA,
        pltpu.SemaphoreType.DMA,
        pltpu.SemaphoreType.DMA,
        pltpu.SemaphoreType.DMA,
        pltpu.SemaphoreType.DMA,
    ],
    compiler_params=pltpu.CompilerParams(use_tc_tiling_on_sc=False, needs_layout_passes=False),
    name="edge_pass_l1",
)

_edge_pass2 = pl.kernel(
    functools.partial(_edge_body, False),
    out_type=jax.ShapeDtypeStruct((NC, N, H), _f32),
    mesh=_sc_mesh,
    scratch_types=[
        pltpu.VMEM((NCHUNK, C), jnp.int32),
        pltpu.VMEM((NCHUNK, C), jnp.int32),
        pltpu.VMEM((C, H), _bf16),
        pltpu.VMEM((C, H), _bf16),
        pltpu.VMEM((C, H), _bf16),
        pltpu.VMEM((C, H), _bf16),
        pltpu.VMEM((C, H), _f32),
        pltpu.VMEM((C, H), _f32),
        pltpu.VMEM((ZR, H), _f32),
        pltpu.VMEM_SHARED((N, H), _f32),
        pltpu.SemaphoreType.DMA,
        pltpu.SemaphoreType.DMA,
        pltpu.SemaphoreType.DMA,
        pltpu.SemaphoreType.DMA,
        pltpu.SemaphoreType.DMA,
        pltpu.SemaphoreType.DMA,
    ],
    compiler_params=pltpu.CompilerParams(use_tc_tiling_on_sc=False, needs_layout_passes=False),
    name="edge_pass_l2",
)


def kernel(x, edge_index, W1a, b1a, W1b, b1b, W2a, b2a, W2b, b2b, Wl, bl):
    src = edge_index[0].astype(jnp.int32).reshape(NW, NCHUNK, C)
    dst = edge_index[1].astype(jnp.int32).reshape(NW, NCHUNK, C)

    w1 = jnp.concatenate([W1a[:D] - W1a[D:], W1a[D:]], axis=1)
    bias1 = jnp.concatenate([b1a, jnp.zeros_like(b1a)])[None]
    w2 = jnp.concatenate([W2a[:H] - W2a[H:], W2a[H:]], axis=1)
    bias2 = jnp.concatenate([b2a, jnp.zeros_like(b2a)])[None]
    wl_pad = jnp.zeros((H, D), _f32).at[:, :O].set(Wl)
    bl_pad = jnp.zeros((1, D), _f32).at[0, :O].set(bl)

    w1b_p = W1b[_PERM, :]
    w2b_p = W2b[_PERM, :]

    a1, b1v = _make_proj(D)(x, w1, bias1)
    parts1, cnts = _edge_pass1(src, dst, a1, b1v)
    a2, b2v = _mid(parts1, cnts, w1b_p, b1b[None], w2, bias2)
    parts2 = _edge_pass2(src, dst, a2, b2v)
    out = _head(parts2, cnts, w2b_p, b2b[None], wl_pad, bl_pad)
    return out[:, :O]


# speedup vs baseline: 20.9386x; 1.0674x over previous
import functools

import numpy as np

import jax
import jax.numpy as jnp
from jax import lax
from jax.experimental import pallas as pl
from jax.experimental.pallas import tpu as pltpu
from jax.experimental.pallas import tpu_sc as plsc

N = 10000
E = 320000
D = 128
H = 64
O = 2

NC = 2
NS = 16
NW = NC * NS
EPW = E // NW
C = 80
NCHUNK = EPW // C
RPT = N // NS
RO = 624
ZR = 125
CW = 16

RB = 1000
GRID = N // RB

_f32 = jnp.float32
_bf16 = jnp.bfloat16

_PERM = np.concatenate(
    [np.concatenate([g * 32 + 2 * np.arange(16), g * 32 + 2 * np.arange(16) + 1])
     for g in range(H // 32)])


def _proj_body(x_ref, w_ref, b_ref, oa_ref, ob_ref):
    r = jnp.dot(x_ref[...], w_ref[...], preferred_element_type=_f32) + b_ref[...]
    oa_ref[...] = r[:, :H].astype(_bf16)
    ob_ref[...] = r[:, H:].astype(_bf16)


def _make_proj(din):
    return pl.pallas_call(
        _proj_body,
        grid=(GRID,),
        in_specs=[
            pl.BlockSpec((RB, din), lambda i: (i, 0)),
            pl.BlockSpec((din, 2 * H), lambda i: (0, 0)),
            pl.BlockSpec((1, 2 * H), lambda i: (0, 0)),
        ],
        out_specs=[pl.BlockSpec((RB, H), lambda i: (i, 0))] * 2,
        out_shape=[jax.ShapeDtypeStruct((N, H), _bf16)] * 2,
        name="edgeconv_proj",
    )


def _mid_body(parts_ref, cnts_ref, wb_ref, bb_ref, w2_ref, b2_ref,
              oa_ref, ob_ref):
    s = parts_ref[0] + parts_ref[1]
    c = cnts_ref[0, :, :1] + cnts_ref[1, :, :1]
    inv = 1.0 / jnp.maximum(c, 1.0)
    ind = jnp.minimum(c, 1.0)
    mean = jnp.dot(s * inv, wb_ref[...], preferred_element_type=_f32) + ind * bb_ref[...]
    h = jnp.maximum(mean, 0.0)
    r = jnp.dot(h, w2_ref[...], preferred_element_type=_f32) + b2_ref[...]
    oa_ref[...] = r[:, :H].astype(_bf16)
    ob_ref[...] = r[:, H:].astype(_bf16)


_mid = pl.pallas_call(
    _mid_body,
    grid=(GRID,),
    in_specs=[
        pl.BlockSpec((NC, RB, H), lambda i: (0, i, 0)),
        pl.BlockSpec((NC, RB, CW), lambda i: (0, i, 0)),
        pl.BlockSpec((H, H), lambda i: (0, 0)),
        pl.BlockSpec((1, H), lambda i: (0, 0)),
        pl.BlockSpec((H, 2 * H), lambda i: (0, 0)),
        pl.BlockSpec((1, 2 * H), lambda i: (0, 0)),
    ],
    out_specs=[pl.BlockSpec((RB, H), lambda i: (i, 0))] * 2,
    out_shape=[jax.ShapeDtypeStruct((N, H), _bf16)] * 2,
    name="edgeconv_mid",
)


def _head_body(parts_ref, cnts_ref, wb_ref, bb_ref, wl_ref, bl_ref, o_ref):
    s = parts_ref[0] + parts_ref[1]
    c = cnts_ref[0, :, :1] + cnts_ref[1, :, :1]
    inv = 1.0 / jnp.maximum(c, 1.0)
    ind = jnp.minimum(c, 1.0)
    mean = jnp.dot(s * inv, wb_ref[...], preferred_element_type=_f32) + ind * bb_ref[...]
    h = jnp.maximum(mean, 0.0)
    o_ref[...] = jnp.dot(h, wl_ref[...], preferred_element_type=_f32) + bl_ref[...]


_head = pl.pallas_call(
    _head_body,
    grid=(GRID,),
    in_specs=[
        pl.BlockSpec((NC, RB, H), lambda i: (0, i, 0)),
        pl.BlockSpec((NC, RB, CW), lambda i: (0, i, 0)),
        pl.BlockSpec((H, H), lambda i: (0, 0)),
        pl.BlockSpec((1, H), lambda i: (0, 0)),
        pl.BlockSpec((H, D), lambda i: (0, 0)),
        pl.BlockSpec((1, D), lambda i: (0, 0)),
    ],
    out_specs=pl.BlockSpec((RB, D), lambda i: (i, 0)),
    out_shape=jax.ShapeDtypeStruct((N, D), _f32),
    name="edgeconv_head",
)


def _edge_body(with_counts, *refs):
    if with_counts:
        (src_h, dst_h, a_h, b_h, parts_h, cnts_h,
         idx_s2, idx_d2, ba0, ba1, ba2, bb0, bb1, bb2, bo0, bo1, bo2,
         zrows, z16, ones_v,
         acc_sh, cnt_sh, sga0, sga1, sga2, sgb0, sgb1, sgb2,
         ssc0, ssc1, ssc2, scnt) = refs
    else:
        (src_h, dst_h, a_h, b_h, parts_h,
         idx_s2, idx_d2, ba0, ba1, ba2, bb0, bb1, bb2, bo0, bo1, bo2,
         zrows,
         acc_sh, sga0, sga1, sga2, sgb0, sgb1, sgb2,
         ssc0, ssc1, ssc2) = refs
    ba, bb, bo = (ba0, ba1, ba2), (bb0, bb1, bb2), (bo0, bo1, bo2)
    sga, sgb, ssc = (sga0, sga1, sga2), (sgb0, sgb1, sgb2), (ssc0, ssc1, ssc2)

    cid = lax.axis_index("c")
    sid = lax.axis_index("s")
    wid = cid * NS + sid

    zv = jnp.zeros((16,), _f32)

    def zrow(r, carry):
        for c4 in range(H // 16):
            zrows[r, pl.ds(c4 * 16, 16)] = zv
        return carry

    lax.fori_loop(0, ZR, zrow, 0)

    for p in range(RPT // ZR):
        pltpu.sync_copy(zrows, acc_sh.at[pl.ds(sid * RPT + p * ZR, ZR)])

    if with_counts:
        def z16row(r, carry):
            z16[r, pl.ds(0, 16)] = zv
            return carry
        lax.fori_loop(0, ZR, z16row, 0)
        for p in range(RPT // ZR):
            pltpu.sync_copy(z16, cnt_sh.at[pl.ds(sid * RPT + p * ZR, ZR)])
        one = jnp.ones((16,), _f32)

        def onerow(r, carry):
            ones_v[r, pl.ds(0, 16)] = one
            return carry
        lax.fori_loop(0, C, onerow, 0)

    pltpu.sync_copy(src_h.at[wid], idx_s2)
    pltpu.sync_copy(dst_h.at[wid], idx_d2)

    plsc.subcore_barrier()

    def issue_g(t, b):
        pltpu.async_copy(a_h.at[idx_d2.at[t]], ba[b], sga[b])
        pltpu.async_copy(b_h.at[idx_s2.at[t]], bb[b], sgb[b])

    def wait_g(b):
        pltpu.make_async_copy(a_h.at[idx_d2.at[0]], ba[b], sga[b]).wait()
        pltpu.make_async_copy(b_h.at[idx_s2.at[0]], bb[b], sgb[b]).wait()

    def wait_sc(b):
        pltpu.make_async_copy(bo[b], acc_sh.at[idx_d2.at[0]], ssc[b]).wait()

    def wait_cnt():
        pltpu.make_async_copy(ones_v, cnt_sh.at[idx_d2.at[0]], scnt).wait()

    def compute(b):
        bab, bbb, bob = ba[b], bb[b], bo[b]

        @plsc.parallel_loop(0, C, unroll=8)
        def _rows(r):
            for g in range(H // 32):
                sl32 = pl.ds(g * 32, 32)
                ae, ao = plsc.unpack(bab[r, sl32], format=plsc.PackFormat.INTERLEAVED)
                be, bo_ = plsc.unpack(bbb[r, sl32], format=plsc.PackFormat.INTERLEAVED)
                bob[r, pl.ds(g * 32, 16)] = jnp.maximum(ae + be, 0.0)
                bob[r, pl.ds(g * 32 + 16, 16)] = jnp.maximum(ao + bo_, 0.0)

    def handle(t, b, do_wait_sc, do_wait_cnt, do_issue_next):
        if do_issue_next:
            issue_g(t + 2, (b + 2) % 3)
        wait_g(b)
        if do_wait_sc:
            wait_sc(b)
        compute(b)
        pltpu.async_copy(bo[b], acc_sh.at[idx_d2.at[t]], ssc[b], add=True)
        if with_counts:
            if do_wait_cnt:
                wait_cnt()
            pltpu.async_copy(ones_v, cnt_sh.at[idx_d2.at[t]], scnt, add=True)

    issue_g(0, 0)
    issue_g(1, 1)
    handle(0, 0, False, False, True)
    handle(1, 1, False, True, True)
    handle(2, 2, False, True, True)

    def steady(t3, carry):
        t = 3 * t3
        handle(t, 0, True, True, True)
        handle(t + 1, 1, True, True, True)
        handle(t + 2, 2, True, True, True)
        return carry

    lax.fori_loop(1, (NCHUNK - 2) // 3, steady, 0)
    handle(NCHUNK - 2, 0, True, True, False)
    handle(NCHUNK - 1, 1, True, True, False)
    wait_sc(2)
    wait_sc(0)
    wait_sc(1)
    if with_counts:
        wait_cnt()

    plsc.subcore_barrier()

    ro = sid * RO
    pltpu.sync_copy(acc_sh.at[pl.ds(ro, RO)], parts_h.at[cid, pl.ds(ro, RO)])
    if with_counts:
        pltpu.sync_copy(cnt_sh.at[pl.ds(ro, RO)], cnts_h.at[cid, pl.ds(ro, RO)])

    @pl.when(sid == NS - 1)
    def _tail():
        tb = NS * RO
        pltpu.sync_copy(acc_sh.at[pl.ds(tb, N - NS * RO)],
                        parts_h.at[cid, pl.ds(tb, N - NS * RO)])
        if with_counts:
            pltpu.sync_copy(cnt_sh.at[pl.ds(tb, N - NS * RO)],
                            cnts_h.at[cid, pl.ds(tb, N - NS * RO)])


_sc_mesh = plsc.VectorSubcoreMesh(core_axis_name="c", subcore_axis_name="s")

_edge_pass1 = pl.kernel(
    functools.partial(_edge_body, True),
    out_type=[
        jax.ShapeDtypeStruct((NC, N, H), _f32),
        jax.ShapeDtypeStruct((NC, N, CW), _f32),
    ],
    mesh=_sc_mesh,
    scratch_types=[
        pltpu.VMEM((NCHUNK, C), jnp.int32),
        pltpu.VMEM((NCHUNK, C), jnp.int32),
        pltpu.VMEM((C, H), _bf16),
        pltpu.VMEM((C, H), _bf16),
        pltpu.VMEM((C, H), _bf16),
        pltpu.VMEM((C, H), _bf16),
        pltpu.VMEM((C, H), _bf16),
        pltpu.VMEM((C, H), _bf16),
        pltpu.VMEM((C, H), _f32),
        pltpu.VMEM((C, H), _f32),
        pltpu.VMEM((C, H), _f32),
        pltpu.VMEM((ZR, H), _f32),
        pltpu.VMEM((ZR, CW), _f32),
        pltpu.VMEM((C, CW), _f32),
        pltpu.VMEM_SHARED((N, H), _f32),
        pltpu.VMEM_SHARED((N, CW), _f32),
    ] + [pltpu.SemaphoreType.DMA] * 10,
    compiler_params=pltpu.CompilerParams(use_tc_tiling_on_sc=False, needs_layout_passes=False),
    name="edge_pass_l1",
)

_edge_pass2 = pl.kernel(
    functools.partial(_edge_body, False),
    out_type=jax.ShapeDtypeStruct((NC, N, H), _f32),
    mesh=_sc_mesh,
    scratch_types=[
        pltpu.VMEM((NCHUNK, C), jnp.int32),
        pltpu.VMEM((NCHUNK, C), jnp.int32),
        pltpu.VMEM((C, H), _bf16),
        pltpu.VMEM((C, H), _bf16),
        pltpu.VMEM((C, H), _bf16),
        pltpu.VMEM((C, H), _bf16),
        pltpu.VMEM((C, H), _bf16),
        pltpu.VMEM((C, H), _bf16),
        pltpu.VMEM((C, H), _f32),
        pltpu.VMEM((C, H), _f32),
        pltpu.VMEM((C, H), _f32),
        pltpu.VMEM((ZR, H), _f32),
        pltpu.VMEM_SHARED((N, H), _f32),
    ] + [pltpu.SemaphoreType.DMA] * 9,
    compiler_params=pltpu.CompilerParams(use_tc_tiling_on_sc=False, needs_layout_passes=False),
    name="edge_pass_l2",
)


def kernel(x, edge_index, W1a, b1a, W1b, b1b, W2a, b2a, W2b, b2b, Wl, bl):
    src = edge_index[0].astype(jnp.int32).reshape(NW, NCHUNK, C)
    dst = edge_index[1].astype(jnp.int32).reshape(NW, NCHUNK, C)

    w1 = jnp.concatenate([W1a[:D] - W1a[D:], W1a[D:]], axis=1)
    bias1 = jnp.concatenate([b1a, jnp.zeros_like(b1a)])[None]
    w2 = jnp.concatenate([W2a[:H] - W2a[H:], W2a[H:]], axis=1)
    bias2 = jnp.concatenate([b2a, jnp.zeros_like(b2a)])[None]
    wl_pad = jnp.zeros((H, D), _f32).at[:, :O].set(Wl)
    bl_pad = jnp.zeros((1, D), _f32).at[0, :O].set(bl)

    w1b_p = W1b[_PERM, :]
    w2b_p = W2b[_PERM, :]

    a1, b1v = _make_proj(D)(x, w1, bias1)
    parts1, cnts = _edge_pass1(src, dst, a1, b1v)
    a2, b2v = _mid(parts1, cnts, w1b_p, b1b[None], w2, bias2)
    parts2 = _edge_pass2(src, dst, a2, b2v)
    out = _head(parts2, cnts, w2b_p, b2b[None], wl_pad, bl_pad)
    return out[:, :O]
